# Initial kernel scaffold; baseline (speedup 1.0000x reference)
import jax
import jax.numpy as jnp
from jax.experimental import pallas as pl


def kernel(prop_feats, idx0, idx1, idx2, W, b):
    raise NotImplementedError("write your pallas kernel here")


# speedup vs baseline: 1.3802x; 1.3802x over previous
import functools

import jax
import jax.numpy as jnp
from jax import lax
from jax.experimental import pallas as pl
from jax.experimental.pallas import tpu as pltpu
from jax.experimental.pallas import tpu_sc as plsc

N_PROPS = 50000
N_ACTS = 100000
D = 256
L = 16
NC, NS = 2, 16
NW = NC * NS
C = 128
ROWS_PER_W = 3200
PAD_ACTS = NW * ROWS_PER_W
N_CHUNKS = ROWS_PER_W // C

MM_ROWS = 2000


def _mm_body(x_ref, w_ref, o0_ref, o1_ref, o2_ref):
    x = x_ref[...]
    o0_ref[...] = jnp.dot(x, w_ref[0:D, :], preferred_element_type=jnp.float32)
    o1_ref[...] = jnp.dot(x, w_ref[D:2 * D, :], preferred_element_type=jnp.float32)
    o2_ref[...] = jnp.dot(x, w_ref[2 * D:3 * D, :], preferred_element_type=jnp.float32)


def _project_tables(prop2d, W):
    return pl.pallas_call(
        _mm_body,
        grid=(N_PROPS // MM_ROWS,),
        in_specs=[
            pl.BlockSpec((MM_ROWS, D), lambda i: (i, 0)),
            pl.BlockSpec((3 * D, D), lambda i: (0, 0)),
        ],
        out_specs=[pl.BlockSpec((MM_ROWS, D), lambda i: (i, 0))] * 3,
        out_shape=[jax.ShapeDtypeStruct((N_PROPS, D), jnp.float32)] * 3,
    )(prop2d, W)


def _sc_body(p0_hbm, p1_hbm, p2_hbm, i0_hbm, i1_hbm, i2_hbm, b_hbm, out_hbm,
             idx0_v, idx1_v, idx2_v, r0, r1, r2, bias_v, sem):
    wid = lax.axis_index("s") * NC + lax.axis_index("c")
    base = wid * ROWS_PER_W
    pltpu.sync_copy(b_hbm, bias_v)
    bias_regs = [bias_v[pl.ds(j * L, L)] for j in range(D // L)]

    def chunk(c, carry):
        off = base + c * C
        pltpu.sync_copy(i0_hbm.at[pl.ds(off, C)], idx0_v)
        pltpu.sync_copy(i1_hbm.at[pl.ds(off, C)], idx1_v)
        pltpu.sync_copy(i2_hbm.at[pl.ds(off, C)], idx2_v)
        cp0 = pltpu.make_async_copy(p0_hbm.at[idx0_v], r0, sem)
        cp1 = pltpu.make_async_copy(p1_hbm.at[idx1_v], r1, sem)
        cp2 = pltpu.make_async_copy(p2_hbm.at[idx2_v], r2, sem)
        cp0.start()
        cp1.start()
        cp2.start()
        cp0.wait()
        cp1.wait()
        cp2.wait()

        def row(rr, rcarry):
            for j in range(D // L):
                sl = pl.ds(j * L, L)
                x = r0[rr, sl] + r1[rr, sl] + r2[rr, sl] + bias_regs[j]
                r0[rr, sl] = jnp.where(x > 0.0, x, jnp.exp(x) - 1.0)
            return rcarry

        lax.fori_loop(0, C, row, 0)
        pltpu.sync_copy(r0, out_hbm.at[pl.ds(off, C)])
        return carry

    lax.fori_loop(0, N_CHUNKS, chunk, 0)


def _gather_combine(p0, p1, p2, i0, i1, i2, b):
    mesh = plsc.VectorSubcoreMesh(core_axis_name="c", subcore_axis_name="s")
    fn = functools.partial(
        pl.kernel,
        out_type=jax.ShapeDtypeStruct((PAD_ACTS, D), jnp.float32),
        mesh=mesh,
        scratch_types=[
            pltpu.VMEM((C,), jnp.int32),
            pltpu.VMEM((C,), jnp.int32),
            pltpu.VMEM((C,), jnp.int32),
            pltpu.VMEM((C, D), jnp.float32),
            pltpu.VMEM((C, D), jnp.float32),
            pltpu.VMEM((C, D), jnp.float32),
            pltpu.VMEM((D,), jnp.float32),
            pltpu.SemaphoreType.DMA,
        ],
    )(_sc_body)
    return fn(p0, p1, p2, i0, i1, i2, b)


def kernel(prop_feats, idx0, idx1, idx2, W, b):
    prop2d = prop_feats.reshape(N_PROPS, D)
    p0, p1, p2 = _project_tables(prop2d, W)
    pad = jnp.zeros((PAD_ACTS - N_ACTS,), jnp.int32)
    i0 = jnp.concatenate([idx0, pad])
    i1 = jnp.concatenate([idx1, pad])
    i2 = jnp.concatenate([idx2, pad])
    out = _gather_combine(p0, p1, p2, i0, i1, i2, b)
    return out[:N_ACTS].reshape(1, N_ACTS, D)


# speedup vs baseline: 2.3226x; 1.6828x over previous
import functools

import jax
import jax.numpy as jnp
from jax import lax
from jax.experimental import pallas as pl
from jax.experimental.pallas import tpu as pltpu
from jax.experimental.pallas import tpu_sc as plsc

N_PROPS = 50000
N_ACTS = 100000
D = 256
L = 16
NC, NS = 2, 16
NW = NC * NS
C = 48
N_CHUNKS = 66
ROWS_PER_W = C * N_CHUNKS
PAD_ACTS = NW * ROWS_PER_W

FULL_MAX = N_ACTS - C
STRAD_OFF = (N_ACTS // C) * C
STRAD_ROWS = N_ACTS - STRAD_OFF

MM_ROWS = 2000


def _mm_body(x_ref, w_ref, o0_ref, o1_ref, o2_ref):
    x = x_ref[...]
    o0_ref[...] = jnp.dot(x, w_ref[0:D, :], preferred_element_type=jnp.float32)
    o1_ref[...] = jnp.dot(x, w_ref[D:2 * D, :], preferred_element_type=jnp.float32)
    o2_ref[...] = jnp.dot(x, w_ref[2 * D:3 * D, :], preferred_element_type=jnp.float32)


def _project_tables(prop2d, W):
    return pl.pallas_call(
        _mm_body,
        grid=(N_PROPS // MM_ROWS,),
        in_specs=[
            pl.BlockSpec((MM_ROWS, D), lambda i: (i, 0)),
            pl.BlockSpec((3 * D, D), lambda i: (0, 0)),
        ],
        out_specs=[pl.BlockSpec((MM_ROWS, D), lambda i: (i, 0))] * 3,
        out_shape=[jax.ShapeDtypeStruct((N_PROPS, D), jnp.float32)] * 3,
    )(prop2d, W)


def _sc_body(p0_hbm, p1_hbm, p2_hbm, i0_hbm, i1_hbm, i2_hbm, b_hbm, out_hbm,
             ia0, ia1, ia2, r0a, r1a, r2a, r0b, r1b, r2b, oa, ob, bias_v,
             sem_a, sem_b, osem_a, osem_b):
    wid = lax.axis_index("s") * NC + lax.axis_index("c")
    base = wid * ROWS_PER_W
    pltpu.sync_copy(b_hbm, bias_v)
    pltpu.sync_copy(i0_hbm.at[pl.ds(base, ROWS_PER_W)], ia0)
    pltpu.sync_copy(i1_hbm.at[pl.ds(base, ROWS_PER_W)], ia1)
    pltpu.sync_copy(i2_hbm.at[pl.ds(base, ROWS_PER_W)], ia2)
    bias_regs = [bias_v[pl.ds(j * L, L)] for j in range(D // L)]

    banks = ((r0a, r1a, r2a, oa, sem_a, osem_a),
             (r0b, r1b, r2b, ob, sem_b, osem_b))

    def gather_cps(c, bk):
        r0, r1, r2, _, sem, _ = banks[bk]
        off = c * C
        return (pltpu.make_async_copy(p0_hbm.at[ia0.at[pl.ds(off, C)]], r0, sem),
                pltpu.make_async_copy(p1_hbm.at[ia1.at[pl.ds(off, C)]], r1, sem),
                pltpu.make_async_copy(p2_hbm.at[ia2.at[pl.ds(off, C)]], r2, sem))

    def gather_start(c, bk):
        for cp in gather_cps(c, bk):
            cp.start()

    def gather_wait(c, bk):
        for cp in gather_cps(c, bk):
            cp.wait()

    def out_ops(c, bk, start):
        _, _, _, o, _, osem = banks[bk]
        og = base + c * C

        @pl.when(og <= FULL_MAX)
        def _():
            cp = pltpu.make_async_copy(o, out_hbm.at[pl.ds(og, C)], osem)
            cp.start() if start else cp.wait()

        @pl.when(og == STRAD_OFF)
        def _():
            cp = pltpu.make_async_copy(o.at[pl.ds(0, STRAD_ROWS)],
                                       out_hbm.at[pl.ds(og, STRAD_ROWS)], osem)
            cp.start() if start else cp.wait()

    def compute(bk):
        r0, r1, r2, o, _, _ = banks[bk]

        def row(i, carry):
            for u in range(2):
                rr = 2 * i + u
                for j in range(D // L):
                    sl = pl.ds(j * L, L)
                    x = r0[rr, sl] + r1[rr, sl] + r2[rr, sl] + bias_regs[j]
                    o[rr, sl] = jnp.where(x > 0.0, x, jnp.exp(x) - 1.0)
            return carry

        lax.fori_loop(0, C // 2, row, 0)

    gather_start(0, 0)
    gather_start(1, 1)

    def pair(i, carry):
        c0 = 2 * i
        c1 = 2 * i + 1

        gather_wait(c0, 0)

        @pl.when(c0 >= 2)
        def _():
            out_ops(c0 - 2, 0, start=False)

        compute(0)
        out_ops(c0, 0, start=True)

        @pl.when(c0 + 2 < N_CHUNKS)
        def _():
            gather_start(c0 + 2, 0)

        gather_wait(c1, 1)

        @pl.when(c1 >= 3)
        def _():
            out_ops(c1 - 2, 1, start=False)

        compute(1)
        out_ops(c1, 1, start=True)

        @pl.when(c1 + 2 < N_CHUNKS)
        def _():
            gather_start(c1 + 2, 1)

        return carry

    lax.fori_loop(0, N_CHUNKS // 2, pair, 0)
    out_ops(N_CHUNKS - 2, 0, start=False)
    out_ops(N_CHUNKS - 1, 1, start=False)


def _gather_combine(p0, p1, p2, i0, i1, i2, b):
    mesh = plsc.VectorSubcoreMesh(core_axis_name="c", subcore_axis_name="s")
    fn = functools.partial(
        pl.kernel,
        out_type=jax.ShapeDtypeStruct((N_ACTS, D), jnp.float32),
        mesh=mesh,
        scratch_types=[
            pltpu.VMEM((ROWS_PER_W,), jnp.int32),
            pltpu.VMEM((ROWS_PER_W,), jnp.int32),
            pltpu.VMEM((ROWS_PER_W,), jnp.int32),
            pltpu.VMEM((C, D), jnp.float32),
            pltpu.VMEM((C, D), jnp.float32),
            pltpu.VMEM((C, D), jnp.float32),
            pltpu.VMEM((C, D), jnp.float32),
            pltpu.VMEM((C, D), jnp.float32),
            pltpu.VMEM((C, D), jnp.float32),
            pltpu.VMEM((C, D), jnp.float32),
            pltpu.VMEM((C, D), jnp.float32),
            pltpu.VMEM((D,), jnp.float32),
            pltpu.SemaphoreType.DMA,
            pltpu.SemaphoreType.DMA,
            pltpu.SemaphoreType.DMA,
            pltpu.SemaphoreType.DMA,
        ],
    )(_sc_body)
    return fn(p0, p1, p2, i0, i1, i2, b)


def kernel(prop_feats, idx0, idx1, idx2, W, b):
    prop2d = prop_feats.reshape(N_PROPS, D)
    p0, p1, p2 = _project_tables(prop2d, W)
    pad = jnp.zeros((PAD_ACTS - N_ACTS,), jnp.int32)
    i0 = jnp.concatenate([idx0, pad])
    i1 = jnp.concatenate([idx1, pad])
    i2 = jnp.concatenate([idx2, pad])
    out = _gather_combine(p0, p1, p2, i0, i1, i2, b)
    return out.reshape(1, N_ACTS, D)
